# baseline (speedup 1.0000x reference)
import jax
import jax.numpy as jnp
import numpy as np
from jax.experimental import pallas as pl

N, P, D = 4, 1152, 64
NPART, NORIENT = 12, 12
H = W = 64
HW = H * W
R = 224
M = 10
SCALE = 300.0
THR = 1e-4

T = 1024
TROWS = T // W
OROWS = TROWS * R // H
C = 90
ROW_SUM = NPART + NORIENT + D
ROW_MASK = ROW_SUM + 1

_XMAP = np.floor((np.arange(R) + 0.5) * W / R).astype(np.int64)
_REP = np.zeros((W, R), np.float32)
_REP[_XMAP, np.arange(R)] = 1.0
_TILEX = np.tile(np.eye(W, dtype=np.float32), (1, TROWS))
_SPREADY = np.kron(np.eye(TROWS, dtype=np.float32),
                   np.ones((1, W), np.float32))
_THR_UP = float(np.nextafter(np.float32(THR), np.float32(np.inf)))
_RPAT = np.floor((np.arange(OROWS) + 0.5) * TROWS / OROWS).astype(np.int64)
_RSTART = [int(np.searchsorted(_RPAT, j)) for j in range(TROWS)]
_RCOUNT = [int(np.sum(_RPAT == j)) for j in range(TROWS)]


def _fused(px_ref, py_ref, gx_ref, gy_ref, attr_ref, rep_ref,
           part_ref, pw_ref, pm_ref, mask_ref, mbp_ref, vis_ref):
    t_idx = pl.program_id(1)
    px = px_ref[0]
    py = py_ref[0]
    gx = gx_ref[...]
    gy = gy_ref[0]
    wx = jnp.exp(-0.5 * SCALE * (gx - px) ** 2)
    wy = jnp.exp(-0.5 * SCALE * (gy - py) ** 2)
    w = jnp.tile(wx, (1, TROWS)) * jnp.repeat(wy, W, axis=1)

    m = jnp.max(w, axis=0, keepdims=True)
    m1 = m
    wm = w
    for _ in range(M - 1):
        wm = jnp.where(wm >= m, -1.0, wm)
        m = jnp.max(wm, axis=0, keepdims=True)
    mw = jnp.where(w >= jnp.maximum(m, _THR_UP), w, 0.0)

    out = jnp.dot(attr_ref[...], mw, preferred_element_type=jnp.float32)

    part = out[0:NPART]
    orient = out[NPART:NPART + NORIENT]
    orient = orient - jnp.max(orient, axis=0, keepdims=True)
    e = jnp.exp(orient)
    orient = e / jnp.sum(e, axis=0, keepdims=True)
    feat = out[NPART + NORIENT:ROW_SUM]
    feat = feat / jnp.sqrt(jnp.sum(feat * feat, axis=0, keepdims=True) + 1e-12)
    mask_bp = jnp.clip(out[ROW_SUM:ROW_SUM + 1], 0.0, 1.0)
    mask = (m1 > THR).astype(jnp.float32)
    pr = jnp.concatenate([part, orient, feat, mask_bp, mask], axis=0)

    mbp_ref[0] = pr[ROW_SUM:ROW_SUM + 1]

    rep = rep_ref[...]
    for j in range(TROWS):
        s, c = _RSTART[j], _RCOUNT[j]
        ej = jnp.dot(pr[:, j * W:(j + 1) * W], rep,
                     preferred_element_type=jnp.float32)
        part_ref[0, :, s:s + c, :] = jnp.broadcast_to(
            ej[0:NPART, None, :], (NPART, c, R))
        pw_ref[0, :, s:s + c, :] = jnp.broadcast_to(
            ej[NPART:NPART + NORIENT, None, :], (NORIENT, c, R))
        pm_ref[0, :, s:s + c, :] = jnp.broadcast_to(
            ej[NPART + NORIENT:ROW_SUM, None, :], (D, c, R))
        mask_ref[0, s:s + c, :] = jnp.broadcast_to(
            ej[ROW_MASK, None, :], (c, R))

    tv = jnp.max(mw, axis=1, keepdims=True)

    @pl.when(t_idx == 0)
    def _():
        vis_ref[0] = tv

    @pl.when(t_idx > 0)
    def _():
        vis_ref[0] = jnp.maximum(vis_ref[0], tv)


def kernel(verts, faces, vert_orient_weights, vert_part, features):
    del faces
    f32 = jnp.float32
    z = jnp.maximum(verts[..., 2] + 5.0, 0.5)
    px = (verts[..., 0] / z)[..., None]
    py = (verts[..., 1] / z)[..., None]
    gx = jnp.linspace(-1.0, 1.0, W).reshape(1, W).astype(f32)
    gy = jnp.linspace(-1.0, 1.0, H).reshape(HW // T, 1, TROWS).astype(f32)
    attr = jnp.concatenate(
        [vert_part.T, vert_orient_weights.T, features.T,
         jnp.ones((1, P), f32)], axis=0)
    rep = jnp.asarray(_REP)

    n_t = HW // T
    part, pw, pm, mask, mbp, vis = pl.pallas_call(
        _fused,
        grid=(N, n_t),
        in_specs=[
            pl.BlockSpec((1, P, 1), lambda n, t: (n, 0, 0)),
            pl.BlockSpec((1, P, 1), lambda n, t: (n, 0, 0)),
            pl.BlockSpec((1, W), lambda n, t: (0, 0)),
            pl.BlockSpec((1, 1, TROWS), lambda n, t: (t, 0, 0)),
            pl.BlockSpec((ROW_MASK, P), lambda n, t: (0, 0)),
            pl.BlockSpec((W, R), lambda n, t: (0, 0)),
        ],
        out_specs=[
            pl.BlockSpec((1, NPART, OROWS, R), lambda n, t: (n, 0, t, 0)),
            pl.BlockSpec((1, NORIENT, OROWS, R), lambda n, t: (n, 0, t, 0)),
            pl.BlockSpec((1, D, OROWS, R), lambda n, t: (n, 0, t, 0)),
            pl.BlockSpec((1, OROWS, R), lambda n, t: (n, t, 0)),
            pl.BlockSpec((1, 1, T), lambda n, t: (n, 0, t)),
            pl.BlockSpec((1, P, 1), lambda n, t: (n, 0, 0)),
        ],
        out_shape=[
            jax.ShapeDtypeStruct((N, NPART, R, R), f32),
            jax.ShapeDtypeStruct((N, NORIENT, R, R), f32),
            jax.ShapeDtypeStruct((N, D, R, R), f32),
            jax.ShapeDtypeStruct((N, R, R), f32),
            jax.ShapeDtypeStruct((N, 1, HW), f32),
            jax.ShapeDtypeStruct((N, P, 1), f32),
        ],
    )(px, py, gx, gy, attr, rep)

    mask_bp = mbp.reshape(N, H, W)
    vert_visibility = (vis[:, :, 0] > 0.0).astype(jnp.int32)
    return (mask, mask_bp, part, pm, vert_visibility, pw)


# speedup vs baseline: 1.3416x; 1.3416x over previous
import jax
import jax.numpy as jnp
import numpy as np
from jax.experimental import pallas as pl

N, P, D = 4, 1152, 64
NPART, NORIENT = 12, 12
H = W = 64
HW = H * W
R = 224
M = 10
SCALE = 300.0
THR = 1e-4

T = 2048
TROWS = T // W
OROWS = TROWS * R // H
C = 90
ROW_SUM = NPART + NORIENT + D
ROW_MASK = ROW_SUM + 1

_XMAP = np.floor((np.arange(R) + 0.5) * W / R).astype(np.int64)
_REP = np.zeros((W, R), np.float32)
_REP[_XMAP, np.arange(R)] = 1.0
_TILEX = np.tile(np.eye(W, dtype=np.float32), (1, TROWS))
_SPREADY = np.kron(np.eye(TROWS, dtype=np.float32),
                   np.ones((1, W), np.float32))
_THR_UP = float(np.nextafter(np.float32(THR), np.float32(np.inf)))
_RPAT = np.floor((np.arange(OROWS) + 0.5) * TROWS / OROWS).astype(np.int64)
_RSTART = [int(np.searchsorted(_RPAT, j)) for j in range(TROWS)]
_RCOUNT = [int(np.sum(_RPAT == j)) for j in range(TROWS)]


def _fused(px_ref, py_ref, gx_ref, gy_ref, attr_ref, rep_ref,
           part_ref, pw_ref, pm_ref, mask_ref, mbp_ref, vis_ref):
    t_idx = pl.program_id(1)
    px = px_ref[0]
    py = py_ref[0]
    gx = gx_ref[...]
    gy = gy_ref[...]
    d2 = (gx - px) ** 2 + (gy - py) ** 2
    w = jnp.exp(-0.5 * SCALE * d2)

    m = jnp.max(w, axis=0, keepdims=True)
    m1 = m
    wm = w
    for _ in range(M - 1):
        wm = jnp.where(wm >= m, -1.0, wm)
        m = jnp.max(wm, axis=0, keepdims=True)
    mw = jnp.where(w >= jnp.maximum(m, _THR_UP), w, 0.0)

    out = jnp.dot(attr_ref[...], mw, preferred_element_type=jnp.float32)

    part = out[0:NPART]
    orient = out[NPART:NPART + NORIENT]
    orient = orient - jnp.max(orient, axis=0, keepdims=True)
    e = jnp.exp(orient)
    orient = e / jnp.sum(e, axis=0, keepdims=True)
    feat = out[NPART + NORIENT:ROW_SUM]
    feat = feat / jnp.sqrt(jnp.sum(feat * feat, axis=0, keepdims=True) + 1e-12)
    mask_bp = jnp.clip(out[ROW_SUM:ROW_SUM + 1], 0.0, 1.0)
    mask = (m1 > THR).astype(jnp.float32)
    pr = jnp.concatenate([part, orient, feat, mask_bp, mask], axis=0)

    mbp_ref[0] = pr[ROW_SUM:ROW_SUM + 1]

    rep = rep_ref[...]
    for j in range(TROWS):
        s, c = _RSTART[j], _RCOUNT[j]
        ej = jnp.dot(pr[:, j * W:(j + 1) * W], rep,
                     preferred_element_type=jnp.float32)
        part_ref[0, :, s:s + c, :] = jnp.broadcast_to(
            ej[0:NPART, None, :], (NPART, c, R))
        pw_ref[0, :, s:s + c, :] = jnp.broadcast_to(
            ej[NPART:NPART + NORIENT, None, :], (NORIENT, c, R))
        pm_ref[0, :, s:s + c, :] = jnp.broadcast_to(
            ej[NPART + NORIENT:ROW_SUM, None, :], (D, c, R))
        mask_ref[0, s:s + c, :] = jnp.broadcast_to(
            ej[ROW_MASK, None, :], (c, R))

    tv = jnp.max(mw, axis=1, keepdims=True)

    @pl.when(t_idx == 0)
    def _():
        vis_ref[0] = tv

    @pl.when(t_idx > 0)
    def _():
        vis_ref[0] = jnp.maximum(vis_ref[0], tv)


def kernel(verts, faces, vert_orient_weights, vert_part, features):
    del faces
    f32 = jnp.float32
    z = jnp.maximum(verts[..., 2] + 5.0, 0.5)
    px = (verts[..., 0] / z)[..., None]
    py = (verts[..., 1] / z)[..., None]
    xs = jnp.linspace(-1.0, 1.0, W)
    ys = jnp.linspace(-1.0, 1.0, H)
    gyg, gxg = jnp.meshgrid(ys, xs, indexing='ij')
    gx = gxg.reshape(1, HW).astype(f32)
    gy = gyg.reshape(1, HW).astype(f32)
    attr = jnp.concatenate(
        [vert_part.T, vert_orient_weights.T, features.T,
         jnp.ones((1, P), f32)], axis=0)
    rep = jnp.asarray(_REP)

    n_t = HW // T
    part, pw, pm, mask, mbp, vis = pl.pallas_call(
        _fused,
        grid=(N, n_t),
        in_specs=[
            pl.BlockSpec((1, P, 1), lambda n, t: (n, 0, 0)),
            pl.BlockSpec((1, P, 1), lambda n, t: (n, 0, 0)),
            pl.BlockSpec((1, T), lambda n, t: (0, t)),
            pl.BlockSpec((1, T), lambda n, t: (0, t)),
            pl.BlockSpec((ROW_MASK, P), lambda n, t: (0, 0)),
            pl.BlockSpec((W, R), lambda n, t: (0, 0)),
        ],
        out_specs=[
            pl.BlockSpec((1, NPART, OROWS, R), lambda n, t: (n, 0, t, 0)),
            pl.BlockSpec((1, NORIENT, OROWS, R), lambda n, t: (n, 0, t, 0)),
            pl.BlockSpec((1, D, OROWS, R), lambda n, t: (n, 0, t, 0)),
            pl.BlockSpec((1, OROWS, R), lambda n, t: (n, t, 0)),
            pl.BlockSpec((1, 1, T), lambda n, t: (n, 0, t)),
            pl.BlockSpec((1, P, 1), lambda n, t: (n, 0, 0)),
        ],
        out_shape=[
            jax.ShapeDtypeStruct((N, NPART, R, R), f32),
            jax.ShapeDtypeStruct((N, NORIENT, R, R), f32),
            jax.ShapeDtypeStruct((N, D, R, R), f32),
            jax.ShapeDtypeStruct((N, R, R), f32),
            jax.ShapeDtypeStruct((N, 1, HW), f32),
            jax.ShapeDtypeStruct((N, P, 1), f32),
        ],
    )(px, py, gx, gy, attr, rep)

    mask_bp = mbp.reshape(N, H, W)
    vert_visibility = (vis[:, :, 0] > 0.0).astype(jnp.int32)
    return (mask, mask_bp, part, pm, vert_visibility, pw)


# speedup vs baseline: 1.3422x; 1.0004x over previous
import jax
import jax.numpy as jnp
import numpy as np
from jax.experimental import pallas as pl

N, P, D = 4, 1152, 64
NPART, NORIENT = 12, 12
H = W = 64
HW = H * W
R = 224
M = 10
SCALE = 300.0
THR = 1e-4

T = 2048
TROWS = T // W
OROWS = TROWS * R // H
C = 90
ROW_SUM = NPART + NORIENT + D
ROW_MASK = ROW_SUM + 1

_XMAP = np.floor((np.arange(R) + 0.5) * W / R).astype(np.int64)
_REP = np.zeros((W, R), np.float32)
_REP[_XMAP, np.arange(R)] = 1.0
_TILEX = np.tile(np.eye(W, dtype=np.float32), (1, TROWS))
_SPREADY = np.kron(np.eye(TROWS, dtype=np.float32),
                   np.ones((1, W), np.float32))
_THR_UP = float(np.nextafter(np.float32(THR), np.float32(np.inf)))
_RPAT = np.floor((np.arange(OROWS) + 0.5) * TROWS / OROWS).astype(np.int64)
_RSTART = [int(np.searchsorted(_RPAT, j)) for j in range(TROWS)]
_RCOUNT = [int(np.sum(_RPAT == j)) for j in range(TROWS)]


def _fused(px_ref, py_ref, gx_ref, gy_ref, attr_ref, rep_ref,
           part_ref, pw_ref, pm_ref, mask_ref, mbp_ref, vis_ref):
    t_idx = pl.program_id(1)
    px = px_ref[0]
    py = py_ref[0]
    gx = gx_ref[...]
    gy = gy_ref[...]
    d2 = (gx - px) ** 2 + (gy - py) ** 2
    w = jnp.exp(-0.5 * SCALE * d2)

    m = jnp.max(w, axis=0, keepdims=True)
    m1 = m
    for _ in range(M - 1):
        m = jnp.max(jnp.where(w < m, w, -1.0), axis=0, keepdims=True)
    mw = jnp.where(w >= jnp.maximum(m, _THR_UP), w, 0.0)

    out = jnp.dot(attr_ref[...], mw, preferred_element_type=jnp.float32)

    part = out[0:NPART]
    orient = out[NPART:NPART + NORIENT]
    orient = orient - jnp.max(orient, axis=0, keepdims=True)
    e = jnp.exp(orient)
    orient = e / jnp.sum(e, axis=0, keepdims=True)
    feat = out[NPART + NORIENT:ROW_SUM]
    feat = feat / jnp.sqrt(jnp.sum(feat * feat, axis=0, keepdims=True) + 1e-12)
    mask_bp = jnp.clip(out[ROW_SUM:ROW_SUM + 1], 0.0, 1.0)
    mask = (m1 > THR).astype(jnp.float32)
    pr = jnp.concatenate([part, orient, feat, mask_bp, mask], axis=0)

    mbp_ref[0] = pr[ROW_SUM:ROW_SUM + 1]

    rep = rep_ref[...]
    for j in range(TROWS):
        s, c = _RSTART[j], _RCOUNT[j]
        ej = jnp.dot(pr[:, j * W:(j + 1) * W], rep,
                     preferred_element_type=jnp.float32)
        part_ref[0, :, s:s + c, :] = jnp.broadcast_to(
            ej[0:NPART, None, :], (NPART, c, R))
        pw_ref[0, :, s:s + c, :] = jnp.broadcast_to(
            ej[NPART:NPART + NORIENT, None, :], (NORIENT, c, R))
        pm_ref[0, :, s:s + c, :] = jnp.broadcast_to(
            ej[NPART + NORIENT:ROW_SUM, None, :], (D, c, R))
        mask_ref[0, s:s + c, :] = jnp.broadcast_to(
            ej[ROW_MASK, None, :], (c, R))

    tv = jnp.max(mw, axis=1, keepdims=True)

    @pl.when(t_idx == 0)
    def _():
        vis_ref[0] = tv

    @pl.when(t_idx > 0)
    def _():
        vis_ref[0] = jnp.maximum(vis_ref[0], tv)


def kernel(verts, faces, vert_orient_weights, vert_part, features):
    del faces
    f32 = jnp.float32
    z = jnp.maximum(verts[..., 2] + 5.0, 0.5)
    px = (verts[..., 0] / z)[..., None]
    py = (verts[..., 1] / z)[..., None]
    xs = jnp.linspace(-1.0, 1.0, W)
    ys = jnp.linspace(-1.0, 1.0, H)
    gyg, gxg = jnp.meshgrid(ys, xs, indexing='ij')
    gx = gxg.reshape(1, HW).astype(f32)
    gy = gyg.reshape(1, HW).astype(f32)
    attr = jnp.concatenate(
        [vert_part.T, vert_orient_weights.T, features.T,
         jnp.ones((1, P), f32)], axis=0)
    rep = jnp.asarray(_REP)

    n_t = HW // T
    part, pw, pm, mask, mbp, vis = pl.pallas_call(
        _fused,
        grid=(N, n_t),
        in_specs=[
            pl.BlockSpec((1, P, 1), lambda n, t: (n, 0, 0)),
            pl.BlockSpec((1, P, 1), lambda n, t: (n, 0, 0)),
            pl.BlockSpec((1, T), lambda n, t: (0, t)),
            pl.BlockSpec((1, T), lambda n, t: (0, t)),
            pl.BlockSpec((ROW_MASK, P), lambda n, t: (0, 0)),
            pl.BlockSpec((W, R), lambda n, t: (0, 0)),
        ],
        out_specs=[
            pl.BlockSpec((1, NPART, OROWS, R), lambda n, t: (n, 0, t, 0)),
            pl.BlockSpec((1, NORIENT, OROWS, R), lambda n, t: (n, 0, t, 0)),
            pl.BlockSpec((1, D, OROWS, R), lambda n, t: (n, 0, t, 0)),
            pl.BlockSpec((1, OROWS, R), lambda n, t: (n, t, 0)),
            pl.BlockSpec((1, 1, T), lambda n, t: (n, 0, t)),
            pl.BlockSpec((1, P, 1), lambda n, t: (n, 0, 0)),
        ],
        out_shape=[
            jax.ShapeDtypeStruct((N, NPART, R, R), f32),
            jax.ShapeDtypeStruct((N, NORIENT, R, R), f32),
            jax.ShapeDtypeStruct((N, D, R, R), f32),
            jax.ShapeDtypeStruct((N, R, R), f32),
            jax.ShapeDtypeStruct((N, 1, HW), f32),
            jax.ShapeDtypeStruct((N, P, 1), f32),
        ],
    )(px, py, gx, gy, attr, rep)

    mask_bp = mbp.reshape(N, H, W)
    vert_visibility = (vis[:, :, 0] > 0.0).astype(jnp.int32)
    return (mask, mask_bp, part, pm, vert_visibility, pw)


# speedup vs baseline: 1.3646x; 1.0167x over previous
import jax
import jax.numpy as jnp
import numpy as np
from jax.experimental import pallas as pl

N, P, D = 4, 1152, 64
NPART, NORIENT = 12, 12
H = W = 64
HW = H * W
R = 224
M = 10
SCALE = 300.0
THR = 1e-4

T = 2048
TROWS = T // W
OROWS = TROWS * R // H
C = 90
ROW_SUM = NPART + NORIENT + D
ROW_MASK = ROW_SUM + 1

_XMAP = np.floor((np.arange(R) + 0.5) * W / R).astype(np.int64)
_REP = np.zeros((W, R), np.float32)
_REP[_XMAP, np.arange(R)] = 1.0
_TILEX = np.tile(np.eye(W, dtype=np.float32), (1, TROWS))
_SPREADY = np.kron(np.eye(TROWS, dtype=np.float32),
                   np.ones((1, W), np.float32))
_THR_UP = float(np.nextafter(np.float32(THR), np.float32(np.inf)))
_RPAT = np.floor((np.arange(OROWS) + 0.5) * TROWS / OROWS).astype(np.int64)
_RSTART = [int(np.searchsorted(_RPAT, j)) for j in range(TROWS)]
_RCOUNT = [int(np.sum(_RPAT == j)) for j in range(TROWS)]


def _fused(px_ref, py_ref, gx_ref, gy_ref, attr_ref, rep_ref,
           part_ref, pw_ref, pm_ref, mask_ref, mbp_ref, vis_ref):
    t_idx = pl.program_id(1)
    px = px_ref[0]
    py = py_ref[0]
    gx = gx_ref[...]
    gy = gy_ref[...]
    d2 = (gx - px) ** 2 + (gy - py) ** 2
    w = jnp.exp2(-d2)

    m = jnp.max(w, axis=0, keepdims=True)
    m1 = m
    for _ in range(M - 1):
        m = jnp.max(jnp.where(w < m, w, -1.0), axis=0, keepdims=True)
    mw = jnp.where(w >= jnp.maximum(m, _THR_UP), w, 0.0)

    out = jnp.dot(attr_ref[...], mw, preferred_element_type=jnp.float32)

    part = out[0:NPART]
    orient = out[NPART:NPART + NORIENT]
    orient = orient - jnp.max(orient, axis=0, keepdims=True)
    e = jnp.exp(orient)
    orient = e / jnp.sum(e, axis=0, keepdims=True)
    feat = out[NPART + NORIENT:ROW_SUM]
    feat = feat / jnp.sqrt(jnp.sum(feat * feat, axis=0, keepdims=True) + 1e-12)
    mask_bp = jnp.clip(out[ROW_SUM:ROW_SUM + 1], 0.0, 1.0)
    mask = (m1 > THR).astype(jnp.float32)
    pr = jnp.concatenate([part, orient, feat, mask_bp, mask], axis=0)

    mbp_ref[0] = pr[ROW_SUM:ROW_SUM + 1]

    rep = rep_ref[...]
    for j in range(TROWS):
        s, c = _RSTART[j], _RCOUNT[j]
        ej = jnp.dot(pr[:, j * W:(j + 1) * W], rep,
                     preferred_element_type=jnp.float32)
        part_ref[0, :, s:s + c, :] = jnp.broadcast_to(
            ej[0:NPART, None, :], (NPART, c, R))
        pw_ref[0, :, s:s + c, :] = jnp.broadcast_to(
            ej[NPART:NPART + NORIENT, None, :], (NORIENT, c, R))
        pm_ref[0, :, s:s + c, :] = jnp.broadcast_to(
            ej[NPART + NORIENT:ROW_SUM, None, :], (D, c, R))
        mask_ref[0, s:s + c, :] = jnp.broadcast_to(
            ej[ROW_MASK, None, :], (c, R))

    tv = jnp.max(mw, axis=1, keepdims=True)

    @pl.when(t_idx == 0)
    def _():
        vis_ref[0] = tv

    @pl.when(t_idx > 0)
    def _():
        vis_ref[0] = jnp.maximum(vis_ref[0], tv)


def kernel(verts, faces, vert_orient_weights, vert_part, features):
    del faces
    f32 = jnp.float32
    z = jnp.maximum(verts[..., 2] + 5.0, 0.5)
    s = np.sqrt(0.5 * SCALE * np.log2(np.e)).astype(np.float32)
    px = (s * (verts[..., 0] / z))[..., None]
    py = (s * (verts[..., 1] / z))[..., None]
    xs = s * jnp.linspace(-1.0, 1.0, W)
    ys = s * jnp.linspace(-1.0, 1.0, H)
    gyg, gxg = jnp.meshgrid(ys, xs, indexing='ij')
    gx = gxg.reshape(1, HW).astype(f32)
    gy = gyg.reshape(1, HW).astype(f32)
    attr = jnp.concatenate(
        [vert_part.T, vert_orient_weights.T, features.T,
         jnp.ones((1, P), f32)], axis=0)
    rep = jnp.asarray(_REP)

    n_t = HW // T
    part, pw, pm, mask, mbp, vis = pl.pallas_call(
        _fused,
        grid=(N, n_t),
        in_specs=[
            pl.BlockSpec((1, P, 1), lambda n, t: (n, 0, 0)),
            pl.BlockSpec((1, P, 1), lambda n, t: (n, 0, 0)),
            pl.BlockSpec((1, T), lambda n, t: (0, t)),
            pl.BlockSpec((1, T), lambda n, t: (0, t)),
            pl.BlockSpec((ROW_MASK, P), lambda n, t: (0, 0)),
            pl.BlockSpec((W, R), lambda n, t: (0, 0)),
        ],
        out_specs=[
            pl.BlockSpec((1, NPART, OROWS, R), lambda n, t: (n, 0, t, 0)),
            pl.BlockSpec((1, NORIENT, OROWS, R), lambda n, t: (n, 0, t, 0)),
            pl.BlockSpec((1, D, OROWS, R), lambda n, t: (n, 0, t, 0)),
            pl.BlockSpec((1, OROWS, R), lambda n, t: (n, t, 0)),
            pl.BlockSpec((1, 1, T), lambda n, t: (n, 0, t)),
            pl.BlockSpec((1, P, 1), lambda n, t: (n, 0, 0)),
        ],
        out_shape=[
            jax.ShapeDtypeStruct((N, NPART, R, R), f32),
            jax.ShapeDtypeStruct((N, NORIENT, R, R), f32),
            jax.ShapeDtypeStruct((N, D, R, R), f32),
            jax.ShapeDtypeStruct((N, R, R), f32),
            jax.ShapeDtypeStruct((N, 1, HW), f32),
            jax.ShapeDtypeStruct((N, P, 1), f32),
        ],
    )(px, py, gx, gy, attr, rep)

    mask_bp = mbp.reshape(N, H, W)
    vert_visibility = (vis[:, :, 0] > 0.0).astype(jnp.int32)
    return (mask, mask_bp, part, pm, vert_visibility, pw)
